# Initial kernel scaffold; baseline (speedup 1.0000x reference)
import jax
import jax.numpy as jnp
from jax.experimental import pallas as pl


def kernel(node_features):
    raise NotImplementedError("write your pallas kernel here")


# speedup vs baseline: 27.5734x; 27.5734x over previous
import jax
import jax.numpy as jnp
from jax.experimental import pallas as pl

_K = 8


def _knn_tile_kernel(x_ref, xr_ref, ei_ref, ew_ref, *, n, c, row_tile):
    rt = pl.program_id(1)
    row0 = rt * row_tile

    x = x_ref[0]
    norm = jnp.sqrt(jnp.sum(x * x, axis=1, keepdims=True))
    fn = x / jnp.maximum(norm, 1e-12)

    xr = xr_ref[0]
    rnorm = jnp.sqrt(jnp.sum(xr * xr, axis=1, keepdims=True))
    rows = xr / jnp.maximum(rnorm, 1e-12)
    sim = jax.lax.dot_general(
        rows, fn, (((1,), (1,)), ((), ())),
        preferred_element_type=jnp.float32,
    )

    c_iota = jax.lax.broadcasted_iota(jnp.int32, (row_tile, n), 1)
    r_iota = jax.lax.broadcasted_iota(jnp.int32, (row_tile, n), 0) + row0
    neg_inf = jnp.float32(-jnp.inf)
    sim = jnp.where(r_iota == c_iota, neg_inf, sim)

    for j in range(_K):
        m = jnp.max(sim, axis=1, keepdims=True)
        ismax = sim == m
        idx = jnp.min(jnp.where(ismax, c_iota, n), axis=1, keepdims=True)
        ew_ref[0, :, j] = m[:, 0]
        ei_ref[0, 1, :, j] = idx[:, 0]
        if j + 1 < _K:
            sim = jnp.where(c_iota == idx, neg_inf, sim)

    src = jax.lax.broadcasted_iota(jnp.int32, (row_tile, _K), 0) + row0
    ei_ref[0, 0, :, :] = src


def kernel(node_features):
    b, n, c = node_features.shape
    row_tile = 256
    grid = (b, n // row_tile)

    ei, ew = pl.pallas_call(
        lambda x_ref, xr_ref, ei_ref, ew_ref: _knn_tile_kernel(
            x_ref, xr_ref, ei_ref, ew_ref, n=n, c=c, row_tile=row_tile),
        grid=grid,
        in_specs=[
            pl.BlockSpec((1, n, c), lambda i, j: (i, 0, 0)),
            pl.BlockSpec((1, row_tile, c), lambda i, j: (i, j, 0)),
        ],
        out_specs=[
            pl.BlockSpec((1, 2, row_tile, _K), lambda i, j: (i, 0, j, 0)),
            pl.BlockSpec((1, row_tile, _K), lambda i, j: (i, j, 0)),
        ],
        out_shape=[
            jax.ShapeDtypeStruct((b, 2, n, _K), jnp.int32),
            jax.ShapeDtypeStruct((b, n, _K), jnp.float32),
        ],
    )(node_features, node_features)

    edge_index = ei.reshape(b, 2, n * _K)
    edge_weight = ew.reshape(b, n * _K)
    return edge_index, edge_weight


# speedup vs baseline: 37.2266x; 1.3501x over previous
import jax
import jax.numpy as jnp
from jax.experimental import pallas as pl
from jax.experimental.pallas import tpu as pltpu

_K = 8


def _knn_tile_kernel(x_ref, xr_ref, ei_ref, ew_ref, fn_ref, *, n, c, row_tile):
    rt = pl.program_id(1)
    row0 = rt * row_tile

    @pl.when(rt == 0)
    def _normalize():
        x = x_ref[0]
        norm = jnp.sqrt(jnp.sum(x * x, axis=1, keepdims=True))
        fn_ref[...] = x / jnp.maximum(norm, 1e-12)

    fn = fn_ref[...]
    xr = xr_ref[0]
    rnorm = jnp.sqrt(jnp.sum(xr * xr, axis=1, keepdims=True))
    rows = xr / jnp.maximum(rnorm, 1e-12)
    sim = jax.lax.dot_general(
        rows, fn, (((1,), (1,)), ((), ())),
        preferred_element_type=jnp.float32,
    )

    c_iota_i = jax.lax.broadcasted_iota(jnp.int32, (row_tile, n), 1)
    r_iota_i = jax.lax.broadcasted_iota(jnp.int32, (row_tile, n), 0) + row0
    neg_inf = jnp.float32(-jnp.inf)
    sim = jnp.where(r_iota_i == c_iota_i, neg_inf, sim)
    c_iota = c_iota_i.astype(jnp.float32)

    nf = jnp.float32(n)
    for j in range(_K):
        m = jnp.max(sim, axis=1, keepdims=True)
        ismax = sim == m
        idxf = jnp.min(jnp.where(ismax, c_iota, nf), axis=1, keepdims=True)
        ew_ref[0, :, j] = m[:, 0]
        ei_ref[0, 1, :, j] = idxf[:, 0].astype(jnp.int32)
        if j + 1 < _K:
            sim = jnp.where(c_iota == idxf, neg_inf, sim)

    src = jax.lax.broadcasted_iota(jnp.int32, (row_tile, _K), 0) + row0
    ei_ref[0, 0, :, :] = src


def kernel(node_features):
    b, n, c = node_features.shape
    row_tile = 256
    grid = (b, n // row_tile)

    ei, ew = pl.pallas_call(
        lambda x_ref, xr_ref, ei_ref, ew_ref, fn_ref: _knn_tile_kernel(
            x_ref, xr_ref, ei_ref, ew_ref, fn_ref,
            n=n, c=c, row_tile=row_tile),
        scratch_shapes=[pltpu.VMEM((n, c), jnp.float32)],
        grid=grid,
        in_specs=[
            pl.BlockSpec((1, n, c), lambda i, j: (i, 0, 0)),
            pl.BlockSpec((1, row_tile, c), lambda i, j: (i, j, 0)),
        ],
        out_specs=[
            pl.BlockSpec((1, 2, row_tile, _K), lambda i, j: (i, 0, j, 0)),
            pl.BlockSpec((1, row_tile, _K), lambda i, j: (i, j, 0)),
        ],
        out_shape=[
            jax.ShapeDtypeStruct((b, 2, n, _K), jnp.int32),
            jax.ShapeDtypeStruct((b, n, _K), jnp.float32),
        ],
    )(node_features, node_features)

    edge_index = ei.reshape(b, 2, n * _K)
    edge_weight = ew.reshape(b, n * _K)
    return edge_index, edge_weight


# speedup vs baseline: 41.5323x; 1.1157x over previous
import jax
import jax.numpy as jnp
from jax.experimental import pallas as pl

_K = 8


def _knn_kernel(x_ref, ei_ref, ew_ref, *, n):
    x = x_ref[0]
    norm = jnp.sqrt(jnp.sum(x * x, axis=1, keepdims=True))
    fn = x / jnp.maximum(norm, 1e-12)
    sim = jax.lax.dot_general(
        fn, fn, (((1,), (1,)), ((), ())),
        preferred_element_type=jnp.float32,
    )

    c_iota_i = jax.lax.broadcasted_iota(jnp.int32, (n, n), 1)
    r_iota_i = jax.lax.broadcasted_iota(jnp.int32, (n, n), 0)
    neg_inf = jnp.float32(-jnp.inf)
    sim = jnp.where(r_iota_i == c_iota_i, neg_inf, sim)
    c_iota = c_iota_i.astype(jnp.float32)

    nf = jnp.float32(n)
    for j in range(_K):
        m = jnp.max(sim, axis=1, keepdims=True)
        ismax = sim == m
        idxf = jnp.min(jnp.where(ismax, c_iota, nf), axis=1, keepdims=True)
        ew_ref[0, :, j] = m[:, 0]
        ei_ref[0, 1, :, j] = idxf[:, 0].astype(jnp.int32)
        if j + 1 < _K:
            sim = jnp.where(c_iota == idxf, neg_inf, sim)

    src = jax.lax.broadcasted_iota(jnp.int32, (n, _K), 0)
    ei_ref[0, 0, :, :] = src


def kernel(node_features):
    b, n, c = node_features.shape

    ei, ew = pl.pallas_call(
        lambda x_ref, ei_ref, ew_ref: _knn_kernel(x_ref, ei_ref, ew_ref, n=n),
        grid=(b,),
        in_specs=[pl.BlockSpec((1, n, c), lambda i: (i, 0, 0))],
        out_specs=[
            pl.BlockSpec((1, 2, n, _K), lambda i: (i, 0, 0, 0)),
            pl.BlockSpec((1, n, _K), lambda i: (i, 0, 0)),
        ],
        out_shape=[
            jax.ShapeDtypeStruct((b, 2, n, _K), jnp.int32),
            jax.ShapeDtypeStruct((b, n, _K), jnp.float32),
        ],
    )(node_features)

    edge_index = ei.reshape(b, 2, n * _K)
    edge_weight = ew.reshape(b, n * _K)
    return edge_index, edge_weight
